# baseline (speedup 1.0000x reference)
import functools

import jax
import jax.numpy as jnp
from jax.experimental import pallas as pl
from jax.experimental.pallas import tpu as pltpu

_BIG = 9999999.0
_NLAB = 512


def _mine_kernel(e1_ref, e2_ref, l1_ref, l2_ref, out_ref,
                 e1aug_ref, n1_ref, ofs_ref):
    j = pl.program_id(0)
    nsteps = pl.num_programs(0)
    bm, k = e1_ref.shape
    bn = e2_ref.shape[0]

    @pl.when(j == 0)
    def _build_e1aug():
        e1 = e1_ref[...]
        n1 = jnp.sqrt(jnp.sum(e1 * e1, axis=1, keepdims=True)) + 1e-12
        ofs_b = (256.0 * n1).astype(jnp.bfloat16)
        c1 = jax.lax.broadcasted_iota(jnp.int32, (bm, _NLAB), 1)
        e1aug_ref[:, :k] = e1.astype(jnp.bfloat16)
        e1aug_ref[:, k:] = jnp.where(
            c1 == l1_ref[...], -ofs_b.astype(jnp.float32), 0.0
        ).astype(jnp.bfloat16)
        n1_ref[0, :] = n1.reshape(bm)
        ofs_ref[0, :] = ofs_b.astype(jnp.float32).reshape(bm)

    e2 = e2_ref[...]
    inv2 = 1.0 / (jnp.sqrt(jnp.sum(e2 * e2, axis=1, keepdims=True)) + 1e-12)
    c2 = jax.lax.broadcasted_iota(jnp.int32, (bn, _NLAB), 1)
    e2aug = jnp.concatenate(
        [(e2 * inv2).astype(jnp.bfloat16),
         (c2 == l2_ref[...]).astype(jnp.float32).astype(jnp.bfloat16)],
        axis=1,
    )

    a = jax.lax.dot_general(
        e2aug, e1aug_ref[...], (((1,), (1,)), ((), ())),
        preferred_element_type=jnp.float32,
    )

    p_tile = jnp.min(a, axis=0)
    n_tile = jnp.max(a, axis=0)

    @pl.when(j == 0)
    def _init():
        out_ref[0, :] = p_tile
        out_ref[1, :] = n_tile

    @pl.when(j != 0)
    def _fold():
        out_ref[0, :] = jnp.minimum(out_ref[0, :], p_tile)
        out_ref[1, :] = jnp.maximum(out_ref[1, :], n_tile)

    @pl.when(j == nsteps - 1)
    def _fixup():
        inv1 = 1.0 / n1_ref[0, :]
        ofs = ofs_ref[0, :]
        thr = -0.5 * ofs
        ps = out_ref[0, :]
        ns = out_ref[1, :]
        out_ref[0, :] = jnp.where(
            ps < thr, (ps + ofs) * inv1, ps * inv1 + _BIG)
        out_ref[1, :] = jnp.where(
            ns < thr, (ns + ofs) * inv1 - _BIG, ns * inv1)


@functools.partial(jax.jit, static_argnames=("bn",))
def _mine(emb1, emb2, label1, label2, bn=1024):
    m, k = emb1.shape
    n = emb2.shape[0]
    l1 = label1.reshape(m, 1)
    l2 = label2.reshape(n, 1)
    grid = (n // bn,)
    return pl.pallas_call(
        _mine_kernel,
        grid=grid,
        in_specs=[
            pl.BlockSpec((m, k), lambda j: (0, 0)),
            pl.BlockSpec((bn, k), lambda j: (j, 0)),
            pl.BlockSpec((m, 1), lambda j: (0, 0)),
            pl.BlockSpec((bn, 1), lambda j: (j, 0)),
        ],
        out_specs=pl.BlockSpec((2, m), lambda j: (0, 0)),
        out_shape=jax.ShapeDtypeStruct((2, m), jnp.float32),
        scratch_shapes=[
            pltpu.VMEM((m, k + _NLAB), jnp.bfloat16),
            pltpu.VMEM((1, m), jnp.float32),
            pltpu.VMEM((1, m), jnp.float32),
        ],
        compiler_params=pltpu.CompilerParams(
            dimension_semantics=("arbitrary",),
        ),
    )(emb1, emb2, l1, l2)


def kernel(emb1, emb2, label1, label2):
    return _mine(emb1, emb2, label1, label2)


# speedup vs baseline: 1.0122x; 1.0122x over previous
import functools

import jax
import jax.numpy as jnp
from jax.experimental import pallas as pl
from jax.experimental.pallas import tpu as pltpu

_BIG = 9999999.0
_NLAB = 512


def _mine_kernel(e1_ref, e2_ref, l1_ref, l2_ref, out_ref,
                 e1b_ref, o1_ref, n1_ref, ofs_ref):
    j = pl.program_id(0)
    nsteps = pl.num_programs(0)
    bm, k = e1_ref.shape
    bn = e2_ref.shape[0]

    @pl.when(j == 0)
    def _build_e1aug():
        e1 = e1_ref[...]
        n1 = jnp.sqrt(jnp.sum(e1 * e1, axis=1, keepdims=True)) + 1e-12
        ofs_b = (256.0 * n1).astype(jnp.bfloat16)
        c1 = jax.lax.broadcasted_iota(jnp.int32, (bm, _NLAB), 1)
        e1b_ref[...] = e1.astype(jnp.bfloat16)
        o1_ref[...] = jnp.where(
            c1 == l1_ref[...], -ofs_b.astype(jnp.float32), 0.0
        ).astype(jnp.bfloat16)
        n1_ref[0, :] = n1.reshape(bm)
        ofs_ref[0, :] = ofs_b.astype(jnp.float32).reshape(bm)

    e2 = e2_ref[...]
    inv2 = 1.0 / (jnp.sqrt(jnp.sum(e2 * e2, axis=1, keepdims=True)) + 1e-12)
    e2n = (e2 * inv2).astype(jnp.bfloat16)
    c2 = jax.lax.broadcasted_iota(jnp.int32, (bn, _NLAB), 1)
    o2 = (c2 == l2_ref[...]).astype(jnp.float32).astype(jnp.bfloat16)

    dims = (((1,), (1,)), ((), ()))
    a = jax.lax.dot_general(
        e2n, e1b_ref[...], dims, preferred_element_type=jnp.float32,
    ) + jax.lax.dot_general(
        o2, o1_ref[...], dims, preferred_element_type=jnp.float32,
    )

    p_tile = jnp.min(a, axis=0)
    n_tile = jnp.max(a, axis=0)

    @pl.when(j == 0)
    def _init():
        out_ref[0, :] = p_tile
        out_ref[1, :] = n_tile

    @pl.when(j != 0)
    def _fold():
        out_ref[0, :] = jnp.minimum(out_ref[0, :], p_tile)
        out_ref[1, :] = jnp.maximum(out_ref[1, :], n_tile)

    @pl.when(j == nsteps - 1)
    def _fixup():
        inv1 = 1.0 / n1_ref[0, :]
        ofs = ofs_ref[0, :]
        thr = -0.5 * ofs
        ps = out_ref[0, :]
        ns = out_ref[1, :]
        out_ref[0, :] = jnp.where(
            ps < thr, (ps + ofs) * inv1, ps * inv1 + _BIG)
        out_ref[1, :] = jnp.where(
            ns < thr, (ns + ofs) * inv1 - _BIG, ns * inv1)


@functools.partial(jax.jit, static_argnames=("bn",))
def _mine(emb1, emb2, label1, label2, bn=512):
    m, k = emb1.shape
    n = emb2.shape[0]
    l1 = label1.reshape(m, 1)
    l2 = label2.reshape(n, 1)
    grid = (n // bn,)
    return pl.pallas_call(
        _mine_kernel,
        grid=grid,
        in_specs=[
            pl.BlockSpec((m, k), lambda j: (0, 0)),
            pl.BlockSpec((bn, k), lambda j: (j, 0)),
            pl.BlockSpec((m, 1), lambda j: (0, 0)),
            pl.BlockSpec((bn, 1), lambda j: (j, 0)),
        ],
        out_specs=pl.BlockSpec((2, m), lambda j: (0, 0)),
        out_shape=jax.ShapeDtypeStruct((2, m), jnp.float32),
        scratch_shapes=[
            pltpu.VMEM((m, k), jnp.bfloat16),
            pltpu.VMEM((m, _NLAB), jnp.bfloat16),
            pltpu.VMEM((1, m), jnp.float32),
            pltpu.VMEM((1, m), jnp.float32),
        ],
        compiler_params=pltpu.CompilerParams(
            dimension_semantics=("arbitrary",),
        ),
    )(emb1, emb2, l1, l2)


def kernel(emb1, emb2, label1, label2):
    return _mine(emb1, emb2, label1, label2)
